# Initial kernel scaffold; baseline (speedup 1.0000x reference)
import jax
import jax.numpy as jnp
from jax.experimental import pallas as pl


def kernel(x, src_map, code_pos, code_mask, batch, ei_child, ei_parent, ei_sibling_next, ei_sibling_prev, ei_dfg_next, ei_dfg_prev, params):
    raise NotImplementedError("write your pallas kernel here")


# speedup vs baseline: 1.4290x; 1.4290x over previous
import functools
import numpy as np
import jax
import jax.numpy as jnp
from jax import lax
from jax.experimental import pallas as pl
from jax.experimental.pallas import tpu as pltpu
from jax.experimental.pallas import tpu_sc as plsc

N = 51200; B = 100; D = 256; H = 8; HD = D // H
FF = 2048; L_ATT = 2; L_GNN = 6; NREL = 6
AST_MAX = 512; CODE_MAX = 256; E = 51200
NODE_VOC = 30000; POS_VOC = 1024; SRC_VOC = 5000

NC = 2
NS = 16
NW = NC * NS
LANES = 16

EPW = E // NS
ECH = 128
NCH = EPW // ECH
RPW = N // NS
CCH = 32
NCC = D // CCH


def _sc_mesh():
    return plsc.VectorSubcoreMesh(
        core_axis_name="c", subcore_axis_name="s",
        num_cores=NC, num_subcores=NS)


def _sc_gather(table, idx, M, K=80):
    T, Dt = table.shape
    mpw = M // NW
    C = mpw // K
    assert mpw % K == 0 and M % NW == 0

    @functools.partial(
        pl.kernel,
        out_type=jax.ShapeDtypeStruct((M, Dt), jnp.float32),
        mesh=_sc_mesh(),
        scratch_types=[
            pltpu.VMEM((2, K), jnp.int32),
            pltpu.VMEM((2, K, Dt), jnp.float32),
            pltpu.SemaphoreType.DMA,
            pltpu.SemaphoreType.DMA,
        ],
        name="sc_row_gather",
        compiler_params=pltpu.CompilerParams(use_tc_tiling_on_sc=False),
    )
    def k(table_h, idx_h, out_h, idxv, rows, gsem0, gsem1):
        w = lax.axis_index("s") * NC + lax.axis_index("c")
        base = w * mpw
        sems = (gsem0, gsem1)

        def start(j):
            b = j % 2
            pltpu.sync_copy(idx_h.at[pl.ds(base + j * K, K)], idxv.at[b])
            return pltpu.async_copy(table_h.at[idxv.at[b]], rows.at[b], sems[b])

        d = start(0)
        for j in range(C):
            nxt = start(j + 1) if j + 1 < C else None
            d.wait()
            pltpu.sync_copy(rows.at[j % 2], out_h.at[pl.ds(base + j * K, K)])
            d = nxt

    return k(table, idx)


def _sc_segsum(hv, src8_w, dst_w, ztile):
    @functools.partial(
        pl.kernel,
        out_type=jax.ShapeDtypeStruct((NREL, N, D), jnp.float32),
        mesh=_sc_mesh(),
        scratch_types=[
            pltpu.VMEM_SHARED((N, CCH), jnp.float32),
            pltpu.VMEM((ECH,), jnp.int32),
            pltpu.VMEM((NCH, ECH), jnp.int32),
            pltpu.VMEM((ECH, CCH), jnp.float32),
            pltpu.VMEM((ECH, CCH), jnp.float32),
            pltpu.SemaphoreType.DMA,
        ],
        name="sc_segsum6",
        compiler_params=pltpu.CompilerParams(use_tc_tiling_on_sc=False),
    )
    def k(hv_h, src_h, dst_h, zt_h, out_h, acc, sidx, didx, stage, zbuf, gsem):
        c = lax.axis_index("c")
        s = lax.axis_index("s")
        pltpu.sync_copy(zt_h, zbuf)

        for r in range(NREL):
            pltpu.sync_copy(dst_h.at[r, s], didx)
            for t in range(NCC // NC):
                cc = c * (NCC // NC) + t
                def zrow(z, _):
                    pltpu.sync_copy(zbuf, acc.at[pl.ds(s * RPW + z * ECH, ECH)])
                    return 0
                lax.fori_loop(0, RPW // ECH, zrow, 0)
                plsc.subcore_barrier()

                def chunk(j, _):
                    pltpu.sync_copy(src_h.at[r, s, j], sidx)
                    def addcc(t16, _):
                        sl = pl.ds(t16 * LANES, LANES)
                        sidx[sl] = sidx[sl] + cc
                        return 0
                    lax.fori_loop(0, ECH // LANES, addcc, 0, unroll=8)
                    pltpu.async_copy(hv_h.at[sidx], stage, gsem).wait()
                    pltpu.sync_copy(stage, acc.at[didx.at[j]], add=True)
                    return 0
                lax.fori_loop(0, NCH, chunk, 0)
                plsc.subcore_barrier()
                pltpu.sync_copy(
                    acc.at[pl.ds(s * RPW, RPW)],
                    out_h.at[r, pl.ds(s * RPW, RPW), pl.ds(cc * CCH, CCH)])
                plsc.subcore_barrier()

    return k(hv, src8_w, dst_w, ztile)


def _sc_degs(dst_w, onetile):
    W16 = 16

    @functools.partial(
        pl.kernel,
        out_type=jax.ShapeDtypeStruct((NREL, N, W16), jnp.float32),
        mesh=_sc_mesh(),
        scratch_types=[
            pltpu.VMEM_SHARED((N, W16), jnp.float32),
            pltpu.VMEM((NCH, ECH), jnp.int32),
            pltpu.VMEM((ECH, W16), jnp.float32),
            pltpu.VMEM((ECH, W16), jnp.float32),
        ],
        name="sc_degs",
        compiler_params=pltpu.CompilerParams(use_tc_tiling_on_sc=False),
    )
    def k(dst_h, one_h, out_h, acc, didx, ones, zbuf):
        c = lax.axis_index("c")
        s = lax.axis_index("s")
        pltpu.sync_copy(one_h.at[pl.ds(0, ECH)], ones)
        pltpu.sync_copy(one_h.at[pl.ds(ECH, ECH)], zbuf)

        for t in range(NREL // NC):
            r = c * (NREL // NC) + t
            pltpu.sync_copy(dst_h.at[r, s], didx)
            def zrow(z, _):
                pltpu.sync_copy(zbuf, acc.at[pl.ds(s * RPW + z * ECH, ECH)])
                return 0
            lax.fori_loop(0, RPW // ECH, zrow, 0)
            plsc.subcore_barrier()
            def chunk(j, _):
                pltpu.sync_copy(ones, acc.at[didx.at[j]], add=True)
                return 0
            lax.fori_loop(0, NCH, chunk, 0)
            plsc.subcore_barrier()
            pltpu.sync_copy(acc.at[pl.ds(s * RPW, RPW)],
                            out_h.at[r, pl.ds(s * RPW, RPW), :])
            plsc.subcore_barrier()

    return k(dst_w, onetile)


def _sc_gather_i32(table, idx, M):
    T = table.shape[0]
    mpw = M // NW

    @functools.partial(
        pl.kernel,
        out_type=jax.ShapeDtypeStruct((M,), jnp.int32),
        mesh=_sc_mesh(),
        scratch_types=[
            pltpu.VMEM((T,), jnp.int32),
            pltpu.VMEM((mpw,), jnp.int32),
            pltpu.VMEM((mpw,), jnp.int32),
        ],
        name="sc_gather_i32",
        compiler_params=pltpu.CompilerParams(
            use_tc_tiling_on_sc=False, needs_layout_passes=False),
    )
    def k(table_h, idx_h, out_h, tab, idxv, outv):
        w = lax.axis_index("s") * NC + lax.axis_index("c")
        base = w * mpw
        pltpu.sync_copy(table_h, tab)
        pltpu.sync_copy(idx_h.at[pl.ds(base, mpw)], idxv)

        def body(j, _):
            ii = idxv[pl.ds(j * LANES, LANES)]
            outv[pl.ds(j * LANES, LANES)] = plsc.load_gather(tab, [ii])
            return 0
        lax.fori_loop(0, mpw // LANES, body, 0, unroll=4)
        pltpu.sync_copy(outv, out_h.at[pl.ds(base, mpw)])

    return k(table, idx)


def _tc_transformer(g_all, g_pm, kbias, p):
    scale = float(np.sqrt(float(D)))
    isq = float(1.0 / np.sqrt(HD))

    def _ln(x, g, b):
        mu = jnp.mean(x, -1, keepdims=True)
        xc = x - mu
        v = jnp.mean(xc * xc, -1, keepdims=True)
        return xc * jax.lax.rsqrt(v + 1e-5) * g + b

    def body(ne_ref, mp_ref, np_ref, kb_ref,
             wq, bq, wk, bk, wv, bv, wo, bo, w1, b1, w2, b2,
             l1g, l1b, l2g, l2b, clng, clnb, out_ref):
        q = ne_ref[0] * scale + mp_ref[0] + np_ref[0]
        q = _ln(q, clng[0], clnb[0])
        bias = kb_ref[0]
        for l in range(L_ATT):
            Q = jnp.dot(q, wq[l], preferred_element_type=jnp.float32) + bq[l]
            K = jnp.dot(q, wk[l], preferred_element_type=jnp.float32) + bk[l]
            V = jnp.dot(q, wv[l], preferred_element_type=jnp.float32) + bv[l]
            outs = []
            for h in range(H):
                sl = slice(h * HD, (h + 1) * HD)
                S = lax.dot_general(Q[:, sl], K[:, sl],
                                    (((1,), (1,)), ((), ())),
                                    preferred_element_type=jnp.float32)
                S = S * isq + bias
                m = jnp.max(S, axis=-1, keepdims=True)
                e = jnp.exp(S - m)
                Pm = e / jnp.sum(e, axis=-1, keepdims=True)
                outs.append(jnp.dot(Pm, V[:, sl],
                                    preferred_element_type=jnp.float32))
            o = jnp.concatenate(outs, axis=-1)
            o = jnp.dot(o, wo[l], preferred_element_type=jnp.float32) + bo[l]
            q = _ln(q + o, l1g[l], l1b[l])
            f = jnp.maximum(
                jnp.dot(q, w1[l], preferred_element_type=jnp.float32) + b1[l], 0.0)
            f = jnp.dot(f, w2[l], preferred_element_type=jnp.float32) + b2[l]
            q = _ln(q + f, l2g[l], l2b[l])
        out_ref[0] = q

    g3 = g_all.reshape(-1, CODE_MAX, D)
    pm3 = g_pm.reshape(-1, CODE_MAX, D)
    row = pl.BlockSpec((1, CODE_MAX, D), lambda b: (b, 0, 0))
    wargs = [p["Wq"], p["bq"], p["Wk"], p["bk"], p["Wv"], p["bv"],
             p["Wo"], p["bo"], p["W1"], p["b1"], p["W2"], p["b2"],
             p["ln1_g"], p["ln1_b"], p["ln2_g"], p["ln2_b"],
             p["code_ln_g"].reshape(1, D), p["code_ln_b"].reshape(1, D)]
    wspecs = [pl.BlockSpec(a.shape, lambda b, nd=a.ndim: (0,) * nd)
              for a in wargs]
    return pl.pallas_call(
        body,
        grid=(B,),
        in_specs=[
            pl.BlockSpec((1, CODE_MAX, D), lambda b: (N // CODE_MAX + b, 0, 0)),
            pl.BlockSpec((1, CODE_MAX, D), lambda b: (b, 0, 0)),
            pl.BlockSpec((1, CODE_MAX, D), lambda b: (B + b, 0, 0)),
            pl.BlockSpec((1, 1, CODE_MAX), lambda b: (b, 0, 0)),
        ] + wspecs,
        out_specs=row,
        out_shape=jax.ShapeDtypeStruct((B, CODE_MAX, D), jnp.float32),
        name="tc_transformer",
    )(g3, pm3, pm3, kbias, *wargs)


def _tc_add(a, b):
    RB = 2048

    def body(a_ref, b_ref, o_ref):
        o_ref[...] = a_ref[...] + b_ref[...]

    spec = pl.BlockSpec((RB, D), lambda i: (i, 0))
    return pl.pallas_call(
        body, grid=(N // RB,),
        in_specs=[spec, spec],
        out_specs=spec,
        out_shape=jax.ShapeDtypeStruct((N, D), jnp.float32),
        name="tc_add",
    )(a, b)


def _tc_gnn_layer(h, aggs, deg8, Wr, Wl, bsum, g, b):
    RB = 512

    def body(h_ref, agg_ref, dg_ref, wr, wl, bs, gg, bb, o_ref):
        hh = h_ref[...]
        out = jnp.dot(hh, wr[...], preferred_element_type=jnp.float32)
        inv = 1.0 / jnp.maximum(dg_ref[...], 1.0)
        for r in range(NREL):
            a = agg_ref[r] * inv[:, r:r + 1]
            out = out + jnp.dot(a, wl[r], preferred_element_type=jnp.float32)
        out = jnp.maximum(out + bs[...], 0.0)
        y = hh + out
        mu = jnp.mean(y, -1, keepdims=True)
        yc = y - mu
        v = jnp.mean(yc * yc, -1, keepdims=True)
        o_ref[...] = yc * jax.lax.rsqrt(v + 1e-5) * gg[...] + bb[...]

    spec = pl.BlockSpec((RB, D), lambda i: (i, 0))
    return pl.pallas_call(
        body, grid=(N // RB,),
        in_specs=[
            spec,
            pl.BlockSpec((NREL, RB, D), lambda i: (0, i, 0)),
            pl.BlockSpec((RB, 8), lambda i: (i, 0)),
            pl.BlockSpec((D, D), lambda i: (0, 0)),
            pl.BlockSpec((NREL, D, D), lambda i: (0, 0, 0)),
            pl.BlockSpec((1, D), lambda i: (0, 0)),
            pl.BlockSpec((1, D), lambda i: (0, 0)),
            pl.BlockSpec((1, D), lambda i: (0, 0)),
        ],
        out_specs=spec,
        out_shape=jax.ShapeDtypeStruct((N, D), jnp.float32),
        name="tc_gnn_layer",
    )(h, aggs, deg8, Wr, Wl, bsum.reshape(1, D), g.reshape(1, D), b.reshape(1, D))


def kernel(x, src_map, code_pos, code_mask, batch, ei_child, ei_parent,
           ei_sibling_next, ei_sibling_prev, ei_dfg_next, ei_dfg_prev, params):
    p = params
    i32 = jnp.int32
    x = x.astype(i32); src_map = src_map.astype(i32)
    code_pos = code_pos.astype(i32); batch = batch.astype(i32)

    k = jnp.sum(code_mask.astype(i32))
    code_idx = jnp.nonzero(code_mask, size=N, fill_value=N)[0].astype(i32)
    valid_k = jnp.arange(N, dtype=i32) < k
    cb = jnp.where(valid_k, batch[jnp.clip(code_idx, 0, N - 1)], B).astype(i32)
    br = jnp.arange(B, dtype=i32)
    c_starts = jnp.searchsorted(cb, br, side="left").astype(i32)
    c_ends = jnp.searchsorted(cb, br, side="right").astype(i32)
    c_counts = c_ends - c_starts
    a_starts = jnp.searchsorted(batch, br, side="left").astype(i32)
    a_ends = jnp.searchsorted(batch, br, side="right").astype(i32)
    a_counts = a_ends - a_starts

    pgrid = jnp.arange(CODE_MAX, dtype=i32)[None, :]
    q_valid = pgrid < jnp.minimum(c_counts, CODE_MAX)[:, None]
    src_slot = jnp.where(q_valid, c_starts[:, None] + pgrid, 0).reshape(-1)
    qv = q_valid.reshape(-1)
    ci_of_slot = jnp.clip(code_idx[src_slot], 0, N - 1)
    ex = jnp.where(qv, x[ci_of_slot], NODE_VOC).astype(i32)
    p0 = jnp.where(qv, code_pos[0][ci_of_slot], 2 * POS_VOC).astype(i32)
    p1 = jnp.where(qv, POS_VOC + code_pos[1][ci_of_slot], 2 * POS_VOC).astype(i32)

    ne_z = jnp.concatenate([p["node_emb"], jnp.zeros((16, D), jnp.float32)], 0)
    pm_z = jnp.concatenate([p["mpos_emb"], p["npos_emb"],
                            jnp.zeros((8, D), jnp.float32)], 0)

    ZROW = B * CODE_MAX
    slot = jnp.arange(N, dtype=i32)
    c_raw = slot - c_starts[jnp.clip(cb, 0, B - 1)]
    c_valid = valid_k & (c_raw < CODE_MAX)
    gidx = jnp.clip(cb, 0, B - 1) * CODE_MAX + jnp.clip(c_raw, 0, CODE_MAX - 1)
    pick = jnp.full((N,), ZROW, dtype=i32)
    pick = pick.at[jnp.where(c_valid, code_idx, N)].set(
        jnp.where(c_valid, gidx, ZROW), mode="drop")

    eis = [ei_child, ei_parent, ei_sibling_next, ei_sibling_prev,
           ei_dfg_next, ei_dfg_prev]
    src8_w = jnp.stack([ei[0].astype(i32) * 8 for ei in eis], 0)
    src8_w = src8_w.reshape(NREL, NS, NCH, ECH)
    dst_w = jnp.stack([ei[1].astype(i32) for ei in eis], 0)
    dst_w = dst_w.reshape(NREL, NS, NCH, ECH)

    agrid = jnp.arange(AST_MAX, dtype=i32)[None, :]
    a_ok = agrid < jnp.minimum(a_counts, AST_MAX)[:, None]
    aidx = jnp.where(a_ok, a_starts[:, None] + agrid, N).reshape(-1)
    cidx2 = jnp.where(qv, ci_of_slot, N)
    kbias = jnp.where(q_valid, 0.0, -1e9).astype(jnp.float32).reshape(B, 1, CODE_MAX)

    g_all = _sc_gather(ne_z, jnp.concatenate([x, ex]), N + B * CODE_MAX)
    g_pm = _sc_gather(pm_z, jnp.concatenate([p0, p1]), 2 * B * CODE_MAX)

    code_enc = _tc_transformer(g_all, g_pm, kbias, p)

    code_flat_z = jnp.concatenate(
        [code_enc.reshape(-1, D), jnp.zeros((8, D), jnp.float32)], 0)
    add_rows = _sc_gather(code_flat_z, pick, N)
    h0 = g_all[:N]
    h = _tc_add(h0, add_rows)

    onetile = jnp.concatenate([jnp.ones((ECH, 16), jnp.float32),
                               jnp.zeros((ECH, 16), jnp.float32)], 0)
    ztile = jnp.zeros((ECH, CCH), jnp.float32)
    deg16 = _sc_degs(dst_w, onetile)
    deg8 = jnp.transpose(deg16[:, :, 0], (1, 0))
    deg8 = jnp.concatenate([deg8, jnp.ones((N, 2), jnp.float32)], 1)
    bsums = jnp.sum(p["bl"], 1)
    for l in range(L_GNN):
        aggs = _sc_segsum(h.reshape(N * 8, CCH), src8_w, dst_w, ztile)
        h = _tc_gnn_layer(h, aggs, deg8, p["Wr"][l], p["Wl"][l], bsums[l],
                          p["gn_g"][l], p["gn_b"][l])

    h_z = jnp.concatenate([h, jnp.zeros((8, D), jnp.float32)], 0)
    fin = _sc_gather(h_z, jnp.concatenate([aidx, cidx2]), N + B * CODE_MAX)
    ast_enc = fin[:N].reshape(B, AST_MAX, D)
    ast_code_enc = fin[N:].reshape(B, CODE_MAX, D)
    sm_z = jnp.concatenate([src_map, jnp.zeros((8,), i32)], 0)
    code_src_map = _sc_gather_i32(sm_z, cidx2, B * CODE_MAX).reshape(B, CODE_MAX)
    return ast_enc, ast_code_enc, code_enc, code_src_map


# speedup vs baseline: 1.6773x; 1.1738x over previous
import functools
import numpy as np
import jax
import jax.numpy as jnp
from jax import lax
from jax.experimental import pallas as pl
from jax.experimental.pallas import tpu as pltpu
from jax.experimental.pallas import tpu_sc as plsc

N = 51200; B = 100; D = 256; H = 8; HD = D // H
FF = 2048; L_ATT = 2; L_GNN = 6; NREL = 6
AST_MAX = 512; CODE_MAX = 256; E = 51200
NODE_VOC = 30000; POS_VOC = 1024; SRC_VOC = 5000

NC = 2
NS = 16
NW = NC * NS
LANES = 16

EPW = E // NS
ECH = 128
NCH = EPW // ECH
RPW = N // NS
CCH = 32
NCC = D // CCH


def _sc_mesh():
    return plsc.VectorSubcoreMesh(
        core_axis_name="c", subcore_axis_name="s",
        num_cores=NC, num_subcores=NS)


def _sc_gather(table, idx, M, K=80):
    T, Dt = table.shape
    mpw = M // NW
    C = mpw // K
    assert mpw % K == 0 and M % NW == 0

    @functools.partial(
        pl.kernel,
        out_type=jax.ShapeDtypeStruct((M, Dt), jnp.float32),
        mesh=_sc_mesh(),
        scratch_types=[
            pltpu.VMEM((2, K), jnp.int32),
            pltpu.VMEM((2, K, Dt), jnp.float32),
            pltpu.SemaphoreType.DMA,
            pltpu.SemaphoreType.DMA,
        ],
        name="sc_row_gather",
        compiler_params=pltpu.CompilerParams(use_tc_tiling_on_sc=False),
    )
    def k(table_h, idx_h, out_h, idxv, rows, gsem0, gsem1):
        w = lax.axis_index("s") * NC + lax.axis_index("c")
        base = w * mpw
        sems = (gsem0, gsem1)

        def start(j):
            b = j % 2
            pltpu.sync_copy(idx_h.at[pl.ds(base + j * K, K)], idxv.at[b])
            return pltpu.async_copy(table_h.at[idxv.at[b]], rows.at[b], sems[b])

        d = start(0)
        for j in range(C):
            nxt = start(j + 1) if j + 1 < C else None
            d.wait()
            pltpu.sync_copy(rows.at[j % 2], out_h.at[pl.ds(base + j * K, K)])
            d = nxt

    return k(table, idx)


def _sc_segsum(hv, src8_w, dst_w, ztile):
    @functools.partial(
        pl.kernel,
        out_type=jax.ShapeDtypeStruct((NREL, N, D), jnp.float32),
        mesh=_sc_mesh(),
        scratch_types=[
            pltpu.VMEM_SHARED((N, CCH), jnp.float32),
            pltpu.VMEM((2, ECH), jnp.int32),
            pltpu.VMEM((NCH, ECH), jnp.int32),
            pltpu.VMEM((2, ECH, CCH), jnp.float32),
            pltpu.VMEM((ECH, CCH), jnp.float32),
            pltpu.SemaphoreType.DMA,
            pltpu.SemaphoreType.DMA,
        ],
        name="sc_segsum6",
        compiler_params=pltpu.CompilerParams(use_tc_tiling_on_sc=False),
    )
    def k(hv_h, src_h, dst_h, zt_h, out_h, acc, sidx, didx, stage, zbuf,
          gsem0, gsem1):
        c = lax.axis_index("c")
        s = lax.axis_index("s")
        sems = (gsem0, gsem1)
        pltpu.sync_copy(zt_h, zbuf)

        for r in range(NREL):
            pltpu.sync_copy(dst_h.at[r, s], didx)
            for t in range(NCC // NC):
                cc = c * (NCC // NC) + t

                def gstart(j, b):
                    pltpu.sync_copy(src_h.at[r, s, j], sidx.at[b])
                    def addcc(t16, _):
                        sl = pl.ds(t16 * LANES, LANES)
                        sidx[b, sl] = sidx[b, sl] + cc
                        return 0
                    lax.fori_loop(0, ECH // LANES, addcc, 0, unroll=8)
                    pltpu.async_copy(hv_h.at[sidx.at[b]], stage.at[b], sems[b])

                def zrow(z, _):
                    pltpu.sync_copy(zbuf, acc.at[pl.ds(s * RPW + z * ECH, ECH)])
                    return 0
                lax.fori_loop(0, RPW // ECH, zrow, 0)
                plsc.subcore_barrier()

                gstart(0, 0)
                gstart(1, 1)

                def pipe(i, _):
                    for b in (0, 1):
                        j = 2 * i + b
                        pltpu.make_async_copy(
                            hv_h.at[sidx.at[b]], stage.at[b], sems[b]).wait()
                        pltpu.sync_copy(stage.at[b], acc.at[didx.at[j]],
                                        add=True)
                        @pl.when(j + 2 < NCH)
                        def _():
                            gstart(j + 2, b)
                    return 0
                lax.fori_loop(0, NCH // 2, pipe, 0)
                pltpu.make_async_copy(
                    hv_h.at[sidx.at[0]], stage.at[0], sems[0]).wait()
                pltpu.sync_copy(stage.at[0], acc.at[didx.at[NCH - 1]], add=True)
                plsc.subcore_barrier()
                pltpu.sync_copy(
                    acc.at[pl.ds(s * RPW, RPW)],
                    out_h.at[r, pl.ds(s * RPW, RPW), pl.ds(cc * CCH, CCH)])
                plsc.subcore_barrier()

    return k(hv, src8_w, dst_w, ztile)


def _sc_degs(dst_w, onetile):
    W16 = 16

    @functools.partial(
        pl.kernel,
        out_type=jax.ShapeDtypeStruct((NREL, N, W16), jnp.float32),
        mesh=_sc_mesh(),
        scratch_types=[
            pltpu.VMEM_SHARED((N, W16), jnp.float32),
            pltpu.VMEM((NCH, ECH), jnp.int32),
            pltpu.VMEM((ECH, W16), jnp.float32),
            pltpu.VMEM((ECH, W16), jnp.float32),
        ],
        name="sc_degs",
        compiler_params=pltpu.CompilerParams(use_tc_tiling_on_sc=False),
    )
    def k(dst_h, one_h, out_h, acc, didx, ones, zbuf):
        c = lax.axis_index("c")
        s = lax.axis_index("s")
        pltpu.sync_copy(one_h.at[pl.ds(0, ECH)], ones)
        pltpu.sync_copy(one_h.at[pl.ds(ECH, ECH)], zbuf)

        for t in range(NREL // NC):
            r = c * (NREL // NC) + t
            pltpu.sync_copy(dst_h.at[r, s], didx)
            def zrow(z, _):
                pltpu.sync_copy(zbuf, acc.at[pl.ds(s * RPW + z * ECH, ECH)])
                return 0
            lax.fori_loop(0, RPW // ECH, zrow, 0)
            plsc.subcore_barrier()
            def chunk(j, _):
                pltpu.sync_copy(ones, acc.at[didx.at[j]], add=True)
                return 0
            lax.fori_loop(0, NCH, chunk, 0)
            plsc.subcore_barrier()
            pltpu.sync_copy(acc.at[pl.ds(s * RPW, RPW)],
                            out_h.at[r, pl.ds(s * RPW, RPW), :])
            plsc.subcore_barrier()

    return k(dst_w, onetile)


def _sc_gather_i32(table, idx, M):
    T = table.shape[0]
    mpw = M // NW

    @functools.partial(
        pl.kernel,
        out_type=jax.ShapeDtypeStruct((M,), jnp.int32),
        mesh=_sc_mesh(),
        scratch_types=[
            pltpu.VMEM((T,), jnp.int32),
            pltpu.VMEM((mpw,), jnp.int32),
            pltpu.VMEM((mpw,), jnp.int32),
        ],
        name="sc_gather_i32",
        compiler_params=pltpu.CompilerParams(
            use_tc_tiling_on_sc=False, needs_layout_passes=False),
    )
    def k(table_h, idx_h, out_h, tab, idxv, outv):
        w = lax.axis_index("s") * NC + lax.axis_index("c")
        base = w * mpw
        pltpu.sync_copy(table_h, tab)
        pltpu.sync_copy(idx_h.at[pl.ds(base, mpw)], idxv)

        def body(j, _):
            ii = idxv[pl.ds(j * LANES, LANES)]
            outv[pl.ds(j * LANES, LANES)] = plsc.load_gather(tab, [ii])
            return 0
        lax.fori_loop(0, mpw // LANES, body, 0, unroll=4)
        pltpu.sync_copy(outv, out_h.at[pl.ds(base, mpw)])

    return k(table, idx)


def _tc_transformer(g_all, g_pm, kbias, p):
    scale = float(np.sqrt(float(D)))
    isq = float(1.0 / np.sqrt(HD))

    def _ln(x, g, b):
        mu = jnp.mean(x, -1, keepdims=True)
        xc = x - mu
        v = jnp.mean(xc * xc, -1, keepdims=True)
        return xc * jax.lax.rsqrt(v + 1e-5) * g + b

    def body(ne_ref, mp_ref, np_ref, kb_ref,
             wq, bq, wk, bk, wv, bv, wo, bo, w1, b1, w2, b2,
             l1g, l1b, l2g, l2b, clng, clnb, out_ref):
        q = ne_ref[0] * scale + mp_ref[0] + np_ref[0]
        q = _ln(q, clng[0], clnb[0])
        bias = kb_ref[0]
        for l in range(L_ATT):
            Q = jnp.dot(q, wq[l], preferred_element_type=jnp.float32) + bq[l]
            K = jnp.dot(q, wk[l], preferred_element_type=jnp.float32) + bk[l]
            V = jnp.dot(q, wv[l], preferred_element_type=jnp.float32) + bv[l]
            outs = []
            for h in range(H):
                sl = slice(h * HD, (h + 1) * HD)
                S = lax.dot_general(Q[:, sl], K[:, sl],
                                    (((1,), (1,)), ((), ())),
                                    preferred_element_type=jnp.float32)
                S = S * isq + bias
                m = jnp.max(S, axis=-1, keepdims=True)
                e = jnp.exp(S - m)
                Pm = e / jnp.sum(e, axis=-1, keepdims=True)
                outs.append(jnp.dot(Pm, V[:, sl],
                                    preferred_element_type=jnp.float32))
            o = jnp.concatenate(outs, axis=-1)
            o = jnp.dot(o, wo[l], preferred_element_type=jnp.float32) + bo[l]
            q = _ln(q + o, l1g[l], l1b[l])
            f = jnp.maximum(
                jnp.dot(q, w1[l], preferred_element_type=jnp.float32) + b1[l], 0.0)
            f = jnp.dot(f, w2[l], preferred_element_type=jnp.float32) + b2[l]
            q = _ln(q + f, l2g[l], l2b[l])
        out_ref[0] = q

    g3 = g_all.reshape(-1, CODE_MAX, D)
    pm3 = g_pm.reshape(-1, CODE_MAX, D)
    row = pl.BlockSpec((1, CODE_MAX, D), lambda b: (b, 0, 0))
    wargs = [p["Wq"], p["bq"], p["Wk"], p["bk"], p["Wv"], p["bv"],
             p["Wo"], p["bo"], p["W1"], p["b1"], p["W2"], p["b2"],
             p["ln1_g"], p["ln1_b"], p["ln2_g"], p["ln2_b"],
             p["code_ln_g"].reshape(1, D), p["code_ln_b"].reshape(1, D)]
    wspecs = [pl.BlockSpec(a.shape, lambda b, nd=a.ndim: (0,) * nd)
              for a in wargs]
    return pl.pallas_call(
        body,
        grid=(B,),
        in_specs=[
            pl.BlockSpec((1, CODE_MAX, D), lambda b: (N // CODE_MAX + b, 0, 0)),
            pl.BlockSpec((1, CODE_MAX, D), lambda b: (b, 0, 0)),
            pl.BlockSpec((1, CODE_MAX, D), lambda b: (B + b, 0, 0)),
            pl.BlockSpec((1, 1, CODE_MAX), lambda b: (b, 0, 0)),
        ] + wspecs,
        out_specs=row,
        out_shape=jax.ShapeDtypeStruct((B, CODE_MAX, D), jnp.float32),
        name="tc_transformer",
    )(g3, pm3, pm3, kbias, *wargs)


def _tc_add(a, b):
    RB = 2048

    def body(a_ref, b_ref, o_ref):
        o_ref[...] = a_ref[...] + b_ref[...]

    spec = pl.BlockSpec((RB, D), lambda i: (i, 0))
    return pl.pallas_call(
        body, grid=(N // RB,),
        in_specs=[spec, spec],
        out_specs=spec,
        out_shape=jax.ShapeDtypeStruct((N, D), jnp.float32),
        name="tc_add",
    )(a, b)


def _tc_gnn_layer(h, aggs, deg8, Wr, Wl, bsum, g, b):
    RB = 512

    def body(h_ref, agg_ref, dg_ref, wr, wl, bs, gg, bb, o_ref):
        hh = h_ref[...]
        out = jnp.dot(hh, wr[...], preferred_element_type=jnp.float32)
        inv = 1.0 / jnp.maximum(dg_ref[...], 1.0)
        for r in range(NREL):
            a = agg_ref[r] * inv[:, r:r + 1]
            out = out + jnp.dot(a, wl[r], preferred_element_type=jnp.float32)
        out = jnp.maximum(out + bs[...], 0.0)
        y = hh + out
        mu = jnp.mean(y, -1, keepdims=True)
        yc = y - mu
        v = jnp.mean(yc * yc, -1, keepdims=True)
        o_ref[...] = yc * jax.lax.rsqrt(v + 1e-5) * gg[...] + bb[...]

    spec = pl.BlockSpec((RB, D), lambda i: (i, 0))
    return pl.pallas_call(
        body, grid=(N // RB,),
        in_specs=[
            spec,
            pl.BlockSpec((NREL, RB, D), lambda i: (0, i, 0)),
            pl.BlockSpec((RB, 8), lambda i: (i, 0)),
            pl.BlockSpec((D, D), lambda i: (0, 0)),
            pl.BlockSpec((NREL, D, D), lambda i: (0, 0, 0)),
            pl.BlockSpec((1, D), lambda i: (0, 0)),
            pl.BlockSpec((1, D), lambda i: (0, 0)),
            pl.BlockSpec((1, D), lambda i: (0, 0)),
        ],
        out_specs=spec,
        out_shape=jax.ShapeDtypeStruct((N, D), jnp.float32),
        name="tc_gnn_layer",
    )(h, aggs, deg8, Wr, Wl, bsum.reshape(1, D), g.reshape(1, D), b.reshape(1, D))


def kernel(x, src_map, code_pos, code_mask, batch, ei_child, ei_parent,
           ei_sibling_next, ei_sibling_prev, ei_dfg_next, ei_dfg_prev, params):
    p = params
    i32 = jnp.int32
    x = x.astype(i32); src_map = src_map.astype(i32)
    code_pos = code_pos.astype(i32); batch = batch.astype(i32)

    k = jnp.sum(code_mask.astype(i32))
    code_idx = jnp.nonzero(code_mask, size=N, fill_value=N)[0].astype(i32)
    valid_k = jnp.arange(N, dtype=i32) < k
    cb = jnp.where(valid_k, batch[jnp.clip(code_idx, 0, N - 1)], B).astype(i32)
    br = jnp.arange(B, dtype=i32)
    c_starts = jnp.searchsorted(cb, br, side="left").astype(i32)
    c_ends = jnp.searchsorted(cb, br, side="right").astype(i32)
    c_counts = c_ends - c_starts
    a_starts = jnp.searchsorted(batch, br, side="left").astype(i32)
    a_ends = jnp.searchsorted(batch, br, side="right").astype(i32)
    a_counts = a_ends - a_starts

    pgrid = jnp.arange(CODE_MAX, dtype=i32)[None, :]
    q_valid = pgrid < jnp.minimum(c_counts, CODE_MAX)[:, None]
    src_slot = jnp.where(q_valid, c_starts[:, None] + pgrid, 0).reshape(-1)
    qv = q_valid.reshape(-1)
    ci_of_slot = jnp.clip(code_idx[src_slot], 0, N - 1)
    ex = jnp.where(qv, x[ci_of_slot], NODE_VOC).astype(i32)
    p0 = jnp.where(qv, code_pos[0][ci_of_slot], 2 * POS_VOC).astype(i32)
    p1 = jnp.where(qv, POS_VOC + code_pos[1][ci_of_slot], 2 * POS_VOC).astype(i32)

    ne_z = jnp.concatenate([p["node_emb"], jnp.zeros((16, D), jnp.float32)], 0)
    pm_z = jnp.concatenate([p["mpos_emb"], p["npos_emb"],
                            jnp.zeros((8, D), jnp.float32)], 0)

    ZROW = B * CODE_MAX
    slot = jnp.arange(N, dtype=i32)
    c_raw = slot - c_starts[jnp.clip(cb, 0, B - 1)]
    c_valid = valid_k & (c_raw < CODE_MAX)
    gidx = jnp.clip(cb, 0, B - 1) * CODE_MAX + jnp.clip(c_raw, 0, CODE_MAX - 1)
    pick = jnp.full((N,), ZROW, dtype=i32)
    pick = pick.at[jnp.where(c_valid, code_idx, N)].set(
        jnp.where(c_valid, gidx, ZROW), mode="drop")

    eis = [ei_child, ei_parent, ei_sibling_next, ei_sibling_prev,
           ei_dfg_next, ei_dfg_prev]
    src8_w = jnp.stack([ei[0].astype(i32) * 8 for ei in eis], 0)
    src8_w = src8_w.reshape(NREL, NS, NCH, ECH)
    dst_w = jnp.stack([ei[1].astype(i32) for ei in eis], 0)
    dst_w = dst_w.reshape(NREL, NS, NCH, ECH)

    agrid = jnp.arange(AST_MAX, dtype=i32)[None, :]
    a_ok = agrid < jnp.minimum(a_counts, AST_MAX)[:, None]
    aidx = jnp.where(a_ok, a_starts[:, None] + agrid, N).reshape(-1)
    cidx2 = jnp.where(qv, ci_of_slot, N)
    kbias = jnp.where(q_valid, 0.0, -1e9).astype(jnp.float32).reshape(B, 1, CODE_MAX)

    g_all = _sc_gather(ne_z, jnp.concatenate([x, ex]), N + B * CODE_MAX)
    g_pm = _sc_gather(pm_z, jnp.concatenate([p0, p1]), 2 * B * CODE_MAX)

    code_enc = _tc_transformer(g_all, g_pm, kbias, p)

    code_flat_z = jnp.concatenate(
        [code_enc.reshape(-1, D), jnp.zeros((8, D), jnp.float32)], 0)
    add_rows = _sc_gather(code_flat_z, pick, N)
    h0 = g_all[:N]
    h = _tc_add(h0, add_rows)

    onetile = jnp.concatenate([jnp.ones((ECH, 16), jnp.float32),
                               jnp.zeros((ECH, 16), jnp.float32)], 0)
    ztile = jnp.zeros((ECH, CCH), jnp.float32)
    deg16 = _sc_degs(dst_w, onetile)
    deg8 = jnp.transpose(deg16[:, :, 0], (1, 0))
    deg8 = jnp.concatenate([deg8, jnp.ones((N, 2), jnp.float32)], 1)
    bsums = jnp.sum(p["bl"], 1)
    for l in range(L_GNN):
        aggs = _sc_segsum(h.reshape(N * 8, CCH), src8_w, dst_w, ztile)
        h = _tc_gnn_layer(h, aggs, deg8, p["Wr"][l], p["Wl"][l], bsums[l],
                          p["gn_g"][l], p["gn_b"][l])

    h_z = jnp.concatenate([h, jnp.zeros((8, D), jnp.float32)], 0)
    fin = _sc_gather(h_z, jnp.concatenate([aidx, cidx2]), N + B * CODE_MAX)
    ast_enc = fin[:N].reshape(B, AST_MAX, D)
    ast_code_enc = fin[N:].reshape(B, CODE_MAX, D)
    sm_z = jnp.concatenate([src_map, jnp.zeros((8,), i32)], 0)
    code_src_map = _sc_gather_i32(sm_z, cidx2, B * CODE_MAX).reshape(B, CODE_MAX)
    return ast_enc, ast_code_enc, code_enc, code_src_map


# speedup vs baseline: 1.7010x; 1.0141x over previous
import functools
import numpy as np
import jax
import jax.numpy as jnp
from jax import lax
from jax.experimental import pallas as pl
from jax.experimental.pallas import tpu as pltpu
from jax.experimental.pallas import tpu_sc as plsc

N = 51200; B = 100; D = 256; H = 8; HD = D // H
FF = 2048; L_ATT = 2; L_GNN = 6; NREL = 6
AST_MAX = 512; CODE_MAX = 256; E = 51200
NODE_VOC = 30000; POS_VOC = 1024; SRC_VOC = 5000

NC = 2
NS = 16
NW = NC * NS
LANES = 16

EPW = E // NS
ECH = 128
NCH = EPW // ECH
RPW = N // NS
CCH = 32
NCC = D // CCH


def _sc_mesh():
    return plsc.VectorSubcoreMesh(
        core_axis_name="c", subcore_axis_name="s",
        num_cores=NC, num_subcores=NS)


def _sc_gather(table, idx, M, K=80, G=2):
    T, Dt = table.shape
    mpw = M // NW
    NG = mpw // (K * G)
    assert mpw % (K * G) == 0 and M % NW == 0

    @functools.partial(
        pl.kernel,
        out_type=jax.ShapeDtypeStruct((M, Dt), jnp.float32),
        mesh=_sc_mesh(),
        scratch_types=[
            pltpu.VMEM((2, G, K), jnp.int32),
            pltpu.VMEM((2, G * K, Dt), jnp.float32),
            pltpu.SemaphoreType.DMA,
            pltpu.SemaphoreType.DMA,
            pltpu.SemaphoreType.DMA,
            pltpu.SemaphoreType.DMA,
        ],
        name="sc_row_gather",
        compiler_params=pltpu.CompilerParams(use_tc_tiling_on_sc=False),
    )
    def k(table_h, idx_h, out_h, idxv, stage, gsem0, gsem1, osem0, osem1):
        w = lax.axis_index("s") * NC + lax.axis_index("c")
        base = w * mpw
        gsems = (gsem0, gsem1)
        osems = (osem0, osem1)

        def gath(grp, b, k_):
            j = grp * G + k_
            pltpu.sync_copy(idx_h.at[pl.ds(base + j * K, K)], idxv.at[b, k_])
            return pltpu.async_copy(
                table_h.at[idxv.at[b, k_]],
                stage.at[b, pl.ds(k_ * K, K)], gsems[b])

        outs = [None, None]
        for grp in range(NG):
            b = grp % 2
            if outs[b] is not None:
                outs[b].wait()
            ds = [gath(grp, b, k_) for k_ in range(G)]
            for d in ds:
                d.wait()
            outs[b] = pltpu.async_copy(
                stage.at[b], out_h.at[pl.ds(base + grp * G * K, G * K)],
                osems[b])
        for b in (0, 1):
            if outs[b] is not None:
                outs[b].wait()

    return k(table, idx)


def _sc_segsum(hv, src8_w, dst_w, ztile):
    @functools.partial(
        pl.kernel,
        out_type=jax.ShapeDtypeStruct((NREL, N, D), jnp.float32),
        mesh=_sc_mesh(),
        scratch_types=[
            pltpu.VMEM_SHARED((N, CCH), jnp.float32),
            pltpu.VMEM((2, ECH), jnp.int32),
            pltpu.VMEM((NCH, ECH), jnp.int32),
            pltpu.VMEM((2, ECH, CCH), jnp.float32),
            pltpu.VMEM((ECH, CCH), jnp.float32),
            pltpu.SemaphoreType.DMA,
            pltpu.SemaphoreType.DMA,
        ],
        name="sc_segsum6",
        compiler_params=pltpu.CompilerParams(use_tc_tiling_on_sc=False),
    )
    def k(hv_h, src_h, dst_h, zt_h, out_h, acc, sidx, didx, stage, zbuf,
          gsem0, gsem1):
        c = lax.axis_index("c")
        s = lax.axis_index("s")
        sems = (gsem0, gsem1)
        pltpu.sync_copy(zt_h, zbuf)

        for r in range(NREL):
            pltpu.sync_copy(dst_h.at[r, s], didx)
            for t in range(NCC // NC):
                cc = c * (NCC // NC) + t

                def gstart(j, b):
                    pltpu.sync_copy(src_h.at[r, s, j], sidx.at[b])
                    def addcc(t16, _):
                        sl = pl.ds(t16 * LANES, LANES)
                        sidx[b, sl] = sidx[b, sl] + cc
                        return 0
                    lax.fori_loop(0, ECH // LANES, addcc, 0, unroll=8)
                    pltpu.async_copy(hv_h.at[sidx.at[b]], stage.at[b], sems[b])

                def zrow(z, _):
                    pltpu.sync_copy(zbuf, acc.at[pl.ds(s * RPW + z * ECH, ECH)])
                    return 0
                lax.fori_loop(0, RPW // ECH, zrow, 0)
                plsc.subcore_barrier()

                gstart(0, 0)
                gstart(1, 1)

                def pipe(i, _):
                    for b in (0, 1):
                        j = 2 * i + b
                        pltpu.make_async_copy(
                            hv_h.at[sidx.at[b]], stage.at[b], sems[b]).wait()
                        pltpu.sync_copy(stage.at[b], acc.at[didx.at[j]],
                                        add=True)
                        @pl.when(j + 2 < NCH)
                        def _():
                            gstart(j + 2, b)
                    return 0
                lax.fori_loop(0, NCH // 2, pipe, 0)
                pltpu.make_async_copy(
                    hv_h.at[sidx.at[0]], stage.at[0], sems[0]).wait()
                pltpu.sync_copy(stage.at[0], acc.at[didx.at[NCH - 1]], add=True)
                plsc.subcore_barrier()
                pltpu.sync_copy(
                    acc.at[pl.ds(s * RPW, RPW)],
                    out_h.at[r, pl.ds(s * RPW, RPW), pl.ds(cc * CCH, CCH)])
                plsc.subcore_barrier()

    return k(hv, src8_w, dst_w, ztile)


def _sc_degs(dst_w, onetile):
    W16 = 16

    @functools.partial(
        pl.kernel,
        out_type=jax.ShapeDtypeStruct((NREL, N, W16), jnp.float32),
        mesh=_sc_mesh(),
        scratch_types=[
            pltpu.VMEM_SHARED((N, W16), jnp.float32),
            pltpu.VMEM((NCH, ECH), jnp.int32),
            pltpu.VMEM((ECH, W16), jnp.float32),
            pltpu.VMEM((ECH, W16), jnp.float32),
        ],
        name="sc_degs",
        compiler_params=pltpu.CompilerParams(use_tc_tiling_on_sc=False),
    )
    def k(dst_h, one_h, out_h, acc, didx, ones, zbuf):
        c = lax.axis_index("c")
        s = lax.axis_index("s")
        pltpu.sync_copy(one_h.at[pl.ds(0, ECH)], ones)
        pltpu.sync_copy(one_h.at[pl.ds(ECH, ECH)], zbuf)

        for t in range(NREL // NC):
            r = c * (NREL // NC) + t
            pltpu.sync_copy(dst_h.at[r, s], didx)
            def zrow(z, _):
                pltpu.sync_copy(zbuf, acc.at[pl.ds(s * RPW + z * ECH, ECH)])
                return 0
            lax.fori_loop(0, RPW // ECH, zrow, 0)
            plsc.subcore_barrier()
            def chunk(j, _):
                pltpu.sync_copy(ones, acc.at[didx.at[j]], add=True)
                return 0
            lax.fori_loop(0, NCH, chunk, 0)
            plsc.subcore_barrier()
            pltpu.sync_copy(acc.at[pl.ds(s * RPW, RPW)],
                            out_h.at[r, pl.ds(s * RPW, RPW), :])
            plsc.subcore_barrier()

    return k(dst_w, onetile)


def _sc_gather_i32(table, idx, M):
    T = table.shape[0]
    mpw = M // NW

    @functools.partial(
        pl.kernel,
        out_type=jax.ShapeDtypeStruct((M,), jnp.int32),
        mesh=_sc_mesh(),
        scratch_types=[
            pltpu.VMEM((T,), jnp.int32),
            pltpu.VMEM((mpw,), jnp.int32),
            pltpu.VMEM((mpw,), jnp.int32),
        ],
        name="sc_gather_i32",
        compiler_params=pltpu.CompilerParams(
            use_tc_tiling_on_sc=False, needs_layout_passes=False),
    )
    def k(table_h, idx_h, out_h, tab, idxv, outv):
        w = lax.axis_index("s") * NC + lax.axis_index("c")
        base = w * mpw
        pltpu.sync_copy(table_h, tab)
        pltpu.sync_copy(idx_h.at[pl.ds(base, mpw)], idxv)

        def body(j, _):
            ii = idxv[pl.ds(j * LANES, LANES)]
            outv[pl.ds(j * LANES, LANES)] = plsc.load_gather(tab, [ii])
            return 0
        lax.fori_loop(0, mpw // LANES, body, 0, unroll=4)
        pltpu.sync_copy(outv, out_h.at[pl.ds(base, mpw)])

    return k(table, idx)


def _tc_transformer(g_all, g_pm, kbias, p):
    scale = float(np.sqrt(float(D)))
    isq = float(1.0 / np.sqrt(HD))

    def _ln(x, g, b):
        mu = jnp.mean(x, -1, keepdims=True)
        xc = x - mu
        v = jnp.mean(xc * xc, -1, keepdims=True)
        return xc * jax.lax.rsqrt(v + 1e-5) * g + b

    bf = jnp.bfloat16

    def bdot(a, w):
        return jnp.dot(a.astype(bf), w, preferred_element_type=jnp.float32)

    def body(ne_ref, mp_ref, np_ref, kb_ref,
             wq, bq, wk, bk, wv, bv, wo, bo, w1, b1, w2, b2,
             l1g, l1b, l2g, l2b, clng, clnb, out_ref):
        q = ne_ref[0] * scale + mp_ref[0] + np_ref[0]
        q = _ln(q, clng[0], clnb[0])
        bias = kb_ref[0]
        for l in range(L_ATT):
            qb = q.astype(bf)
            Q = jnp.dot(qb, wq[l], preferred_element_type=jnp.float32) + bq[l]
            K = jnp.dot(qb, wk[l], preferred_element_type=jnp.float32) + bk[l]
            V = jnp.dot(qb, wv[l], preferred_element_type=jnp.float32) + bv[l]
            outs = []
            for h in range(H):
                sl = slice(h * HD, (h + 1) * HD)
                S = lax.dot_general(Q[:, sl].astype(bf), K[:, sl].astype(bf),
                                    (((1,), (1,)), ((), ())),
                                    preferred_element_type=jnp.float32)
                S = S * isq + bias
                m = jnp.max(S, axis=-1, keepdims=True)
                e = jnp.exp(S - m)
                Pm = e / jnp.sum(e, axis=-1, keepdims=True)
                outs.append(lax.dot_general(
                    Pm.astype(bf), V[:, sl].astype(bf),
                    (((1,), (0,)), ((), ())),
                    preferred_element_type=jnp.float32))
            o = jnp.concatenate(outs, axis=-1)
            o = bdot(o, wo[l]) + bo[l]
            q = _ln(q + o, l1g[l], l1b[l])
            f = jnp.maximum(bdot(q, w1[l]) + b1[l], 0.0)
            f = bdot(f, w2[l]) + b2[l]
            q = _ln(q + f, l2g[l], l2b[l])
        out_ref[0] = q

    g3 = g_all.reshape(-1, CODE_MAX, D)
    pm3 = g_pm.reshape(-1, CODE_MAX, D)
    row = pl.BlockSpec((1, CODE_MAX, D), lambda b: (b, 0, 0))
    bfw = jnp.bfloat16
    wargs = [p["Wq"].astype(bfw), p["bq"], p["Wk"].astype(bfw), p["bk"],
             p["Wv"].astype(bfw), p["bv"], p["Wo"].astype(bfw), p["bo"],
             p["W1"].astype(bfw), p["b1"], p["W2"].astype(bfw), p["b2"],
             p["ln1_g"], p["ln1_b"], p["ln2_g"], p["ln2_b"],
             p["code_ln_g"].reshape(1, D), p["code_ln_b"].reshape(1, D)]
    wspecs = [pl.BlockSpec(a.shape, lambda b, nd=a.ndim: (0,) * nd)
              for a in wargs]
    return pl.pallas_call(
        body,
        grid=(B,),
        in_specs=[
            pl.BlockSpec((1, CODE_MAX, D), lambda b: (N // CODE_MAX + b, 0, 0)),
            pl.BlockSpec((1, CODE_MAX, D), lambda b: (b, 0, 0)),
            pl.BlockSpec((1, CODE_MAX, D), lambda b: (B + b, 0, 0)),
            pl.BlockSpec((1, 1, CODE_MAX), lambda b: (b, 0, 0)),
        ] + wspecs,
        out_specs=row,
        out_shape=jax.ShapeDtypeStruct((B, CODE_MAX, D), jnp.float32),
        name="tc_transformer",
    )(g3, pm3, pm3, kbias, *wargs)


def _tc_add(a, b):
    RB = 2048

    def body(a_ref, b_ref, o_ref):
        o_ref[...] = a_ref[...] + b_ref[...]

    spec = pl.BlockSpec((RB, D), lambda i: (i, 0))
    return pl.pallas_call(
        body, grid=(N // RB,),
        in_specs=[spec, spec],
        out_specs=spec,
        out_shape=jax.ShapeDtypeStruct((N, D), jnp.float32),
        name="tc_add",
    )(a, b)


def _tc_gnn_layer(h, aggs, deg8, Wr, Wl, bsum, g, b):
    RB = 512

    bf = jnp.bfloat16

    def body(h_ref, agg_ref, dg_ref, wr, wl, bs, gg, bb, o_ref):
        hh = h_ref[...]
        out = jnp.dot(hh.astype(bf), wr[...], preferred_element_type=jnp.float32)
        inv = 1.0 / jnp.maximum(dg_ref[...], 1.0)
        for r in range(NREL):
            a = (agg_ref[r] * inv[:, r:r + 1]).astype(bf)
            out = out + jnp.dot(a, wl[r], preferred_element_type=jnp.float32)
        out = jnp.maximum(out + bs[...], 0.0)
        y = hh + out
        mu = jnp.mean(y, -1, keepdims=True)
        yc = y - mu
        v = jnp.mean(yc * yc, -1, keepdims=True)
        o_ref[...] = yc * jax.lax.rsqrt(v + 1e-5) * gg[...] + bb[...]

    spec = pl.BlockSpec((RB, D), lambda i: (i, 0))
    return pl.pallas_call(
        body, grid=(N // RB,),
        in_specs=[
            spec,
            pl.BlockSpec((NREL, RB, D), lambda i: (0, i, 0)),
            pl.BlockSpec((RB, 8), lambda i: (i, 0)),
            pl.BlockSpec((D, D), lambda i: (0, 0)),
            pl.BlockSpec((NREL, D, D), lambda i: (0, 0, 0)),
            pl.BlockSpec((1, D), lambda i: (0, 0)),
            pl.BlockSpec((1, D), lambda i: (0, 0)),
            pl.BlockSpec((1, D), lambda i: (0, 0)),
        ],
        out_specs=spec,
        out_shape=jax.ShapeDtypeStruct((N, D), jnp.float32),
        name="tc_gnn_layer",
    )(h, aggs, deg8, Wr.astype(bf), Wl.astype(bf),
      bsum.reshape(1, D), g.reshape(1, D), b.reshape(1, D))


def kernel(x, src_map, code_pos, code_mask, batch, ei_child, ei_parent,
           ei_sibling_next, ei_sibling_prev, ei_dfg_next, ei_dfg_prev, params):
    p = params
    i32 = jnp.int32
    x = x.astype(i32); src_map = src_map.astype(i32)
    code_pos = code_pos.astype(i32); batch = batch.astype(i32)

    k = jnp.sum(code_mask.astype(i32))
    code_idx = jnp.nonzero(code_mask, size=N, fill_value=N)[0].astype(i32)
    valid_k = jnp.arange(N, dtype=i32) < k
    cb = jnp.where(valid_k, batch[jnp.clip(code_idx, 0, N - 1)], B).astype(i32)
    br = jnp.arange(B, dtype=i32)
    c_starts = jnp.searchsorted(cb, br, side="left").astype(i32)
    c_ends = jnp.searchsorted(cb, br, side="right").astype(i32)
    c_counts = c_ends - c_starts
    a_starts = jnp.searchsorted(batch, br, side="left").astype(i32)
    a_ends = jnp.searchsorted(batch, br, side="right").astype(i32)
    a_counts = a_ends - a_starts

    pgrid = jnp.arange(CODE_MAX, dtype=i32)[None, :]
    q_valid = pgrid < jnp.minimum(c_counts, CODE_MAX)[:, None]
    src_slot = jnp.where(q_valid, c_starts[:, None] + pgrid, 0).reshape(-1)
    qv = q_valid.reshape(-1)
    ci_of_slot = jnp.clip(code_idx[src_slot], 0, N - 1)
    ex = jnp.where(qv, x[ci_of_slot], NODE_VOC).astype(i32)
    p0 = jnp.where(qv, code_pos[0][ci_of_slot], 2 * POS_VOC).astype(i32)
    p1 = jnp.where(qv, POS_VOC + code_pos[1][ci_of_slot], 2 * POS_VOC).astype(i32)

    ne_z = jnp.concatenate([p["node_emb"], jnp.zeros((16, D), jnp.float32)], 0)
    pm_z = jnp.concatenate([p["mpos_emb"], p["npos_emb"],
                            jnp.zeros((8, D), jnp.float32)], 0)

    ZROW = B * CODE_MAX
    slot = jnp.arange(N, dtype=i32)
    c_raw = slot - c_starts[jnp.clip(cb, 0, B - 1)]
    c_valid = valid_k & (c_raw < CODE_MAX)
    gidx = jnp.clip(cb, 0, B - 1) * CODE_MAX + jnp.clip(c_raw, 0, CODE_MAX - 1)
    pick = jnp.full((N,), ZROW, dtype=i32)
    pick = pick.at[jnp.where(c_valid, code_idx, N)].set(
        jnp.where(c_valid, gidx, ZROW), mode="drop")

    eis = [ei_child, ei_parent, ei_sibling_next, ei_sibling_prev,
           ei_dfg_next, ei_dfg_prev]
    src8_w = jnp.stack([ei[0].astype(i32) * 8 for ei in eis], 0)
    src8_w = src8_w.reshape(NREL, NS, NCH, ECH)
    dst_w = jnp.stack([ei[1].astype(i32) for ei in eis], 0)
    dst_w = dst_w.reshape(NREL, NS, NCH, ECH)

    agrid = jnp.arange(AST_MAX, dtype=i32)[None, :]
    a_ok = agrid < jnp.minimum(a_counts, AST_MAX)[:, None]
    aidx = jnp.where(a_ok, a_starts[:, None] + agrid, N).reshape(-1)
    cidx2 = jnp.where(qv, ci_of_slot, N)
    kbias = jnp.where(q_valid, 0.0, -1e9).astype(jnp.float32).reshape(B, 1, CODE_MAX)

    g_all = _sc_gather(ne_z, jnp.concatenate([x, ex]), N + B * CODE_MAX)
    g_pm = _sc_gather(pm_z, jnp.concatenate([p0, p1]), 2 * B * CODE_MAX)

    code_enc = _tc_transformer(g_all, g_pm, kbias, p)

    code_flat_z = jnp.concatenate(
        [code_enc.reshape(-1, D), jnp.zeros((8, D), jnp.float32)], 0)
    add_rows = _sc_gather(code_flat_z, pick, N)
    h0 = g_all[:N]
    h = _tc_add(h0, add_rows)

    onetile = jnp.concatenate([jnp.ones((ECH, 16), jnp.float32),
                               jnp.zeros((ECH, 16), jnp.float32)], 0)
    ztile = jnp.zeros((ECH, CCH), jnp.float32)
    deg16 = _sc_degs(dst_w, onetile)
    deg8 = jnp.transpose(deg16[:, :, 0], (1, 0))
    deg8 = jnp.concatenate([deg8, jnp.ones((N, 2), jnp.float32)], 1)
    bsums = jnp.sum(p["bl"], 1)
    for l in range(L_GNN):
        aggs = _sc_segsum(h.reshape(N * 8, CCH), src8_w, dst_w, ztile)
        h = _tc_gnn_layer(h, aggs, deg8, p["Wr"][l], p["Wl"][l], bsums[l],
                          p["gn_g"][l], p["gn_b"][l])

    h_z = jnp.concatenate([h, jnp.zeros((8, D), jnp.float32)], 0)
    fin = _sc_gather(h_z, jnp.concatenate([aidx, cidx2]), N + B * CODE_MAX)
    ast_enc = fin[:N].reshape(B, AST_MAX, D)
    ast_code_enc = fin[N:].reshape(B, CODE_MAX, D)
    sm_z = jnp.concatenate([src_map, jnp.zeros((8,), i32)], 0)
    code_src_map = _sc_gather_i32(sm_z, cidx2, B * CODE_MAX).reshape(B, CODE_MAX)
    return ast_enc, ast_code_enc, code_enc, code_src_map
